# baseline (speedup 1.0000x reference)
import jax
import jax.numpy as jnp
from jax import lax
from jax.experimental import pallas as pl
from jax.experimental.pallas import tpu as pltpu
from jax.experimental.pallas import tpu_sc as plsc

L = 327680
D = 128
NC = 2
NS = 16
NW = NC * NS
TPW = L // NW
CHUNK = 256
NCHUNKS = TPW // CHUNK


def _sc_body(types_hbm, values_hbm, table_hbm, w_hbm, b_hbm, out_hbm,
             types_v, values_v, rows0_v, rows1_v, table_v, w_v, b_v,
             sem0, sem1, sem_in):
    wid = lax.axis_index("s") * NC + lax.axis_index("c")
    base = wid * TPW

    pltpu.async_copy(table_hbm, table_v, sem_in)
    pltpu.async_copy(w_hbm, w_v, sem_in)
    pltpu.async_copy(b_hbm, b_v, sem_in)
    pltpu.async_copy(types_hbm.at[pl.ds(base, TPW)], types_v, sem_in)
    pltpu.async_copy(values_hbm.at[pl.ds(base, TPW)], values_v, sem_in)
    pltpu.make_async_copy(table_hbm, table_v, sem_in).wait()
    pltpu.make_async_copy(w_hbm, w_v, sem_in).wait()
    pltpu.make_async_copy(b_hbm, b_v, sem_in).wait()
    pltpu.make_async_copy(types_hbm.at[pl.ds(base, TPW)], types_v, sem_in).wait()
    pltpu.make_async_copy(values_hbm.at[pl.ds(base, TPW)], values_v, sem_in).wait()

    for r in range(5):
        for j in range(8):
            sl = pl.ds(16 * j, 16)
            table_v[r, sl] = table_v[r, sl] + b_v[sl]

    w_parts = [w_v[pl.ds(16 * j, 16)] for j in range(8)]

    rows_bufs = (rows0_v, rows1_v)
    sems = (sem0, sem1)

    def compute_chunk(ci, rows_buf):
        @plsc.parallel_loop(0, CHUNK // 16, unroll=8)
        def grp_body(g):
            i0 = ci * CHUNK + g * 16
            tvec = types_v[pl.ds(i0, 16)]
            vvec = values_v[pl.ds(i0, 16)]
            ts = [tvec[k] for k in range(16)]
            vws = [[vvec[k] * w_parts[j] for j in range(8)]
                   for k in range(16)]
            for k in range(16):
                rows = [table_v[ts[k], pl.ds(16 * j, 16)] for j in range(8)]
                for j in range(8):
                    rows_buf[g * 16 + k, pl.ds(16 * j, 16)] = (
                        rows[j] + vws[k][j])

    def pair_body(p, carry):
        for b in range(2):
            ci = p * 2 + b

            @pl.when(p > 0)
            def _wait():
                pltpu.make_async_copy(
                    rows_bufs[b],
                    out_hbm.at[pl.ds(base + ci * CHUNK, CHUNK)],
                    sems[b],
                ).wait()

            compute_chunk(ci, rows_bufs[b])
            pltpu.async_copy(
                rows_bufs[b],
                out_hbm.at[pl.ds(base + ci * CHUNK, CHUNK)],
                sems[b],
            )
        return carry

    lax.fori_loop(0, NCHUNKS // 2, pair_body, 0, unroll=False)

    for b in range(2):
        pltpu.make_async_copy(
            rows_bufs[b],
            out_hbm.at[pl.ds(base, CHUNK)],
            sems[b],
        ).wait()


@jax.jit
def kernel(token_types, token_values, token_embedding, value_W, value_b):
    w = value_W[:, 0]
    values = token_values[:, 0]

    mesh = plsc.VectorSubcoreMesh(core_axis_name="c", subcore_axis_name="s")
    sc_fn = pl.kernel(
        _sc_body,
        mesh=mesh,
        out_type=jax.ShapeDtypeStruct((L, D), jnp.float32),
        scratch_types=[
            pltpu.VMEM((TPW,), jnp.int32),
            pltpu.VMEM((TPW,), jnp.float32),
            pltpu.VMEM((CHUNK, D), jnp.float32),
            pltpu.VMEM((CHUNK, D), jnp.float32),
            pltpu.VMEM((5, D), jnp.float32),
            pltpu.VMEM((D,), jnp.float32),
            pltpu.VMEM((D,), jnp.float32),
            pltpu.SemaphoreType.DMA,
            pltpu.SemaphoreType.DMA,
            pltpu.SemaphoreType.DMA,
        ],
    )
    return sc_fn(token_types, values, token_embedding, w, value_b)


# speedup vs baseline: 2.4688x; 2.4688x over previous
import jax
import jax.numpy as jnp
from jax import lax
from jax.experimental import pallas as pl
from jax.experimental.pallas import tpu as pltpu
from jax.experimental.pallas import tpu_sc as plsc

L = 327680
D = 128
NC = 2
NS = 16
NW = NC * NS
TPW = L // NW
CHUNK = 256
NCHUNKS = TPW // CHUNK


def _sc_body(types_hbm, values_hbm, table_hbm, w_hbm, b_hbm, out_hbm,
             types_v, values_v, rows0_v, rows1_v, table_v, w_v, b_v,
             sem0, sem1, sem_in):
    wid = lax.axis_index("s") * NC + lax.axis_index("c")
    base = wid * TPW

    pltpu.async_copy(table_hbm, table_v, sem_in)
    pltpu.async_copy(w_hbm, w_v, sem_in)
    pltpu.async_copy(b_hbm, b_v, sem_in)
    pltpu.async_copy(types_hbm.at[pl.ds(base, TPW)], types_v, sem_in)
    pltpu.async_copy(values_hbm.at[pl.ds(base, TPW)], values_v, sem_in)
    pltpu.make_async_copy(table_hbm, table_v, sem_in).wait()
    pltpu.make_async_copy(w_hbm, w_v, sem_in).wait()
    pltpu.make_async_copy(b_hbm, b_v, sem_in).wait()
    pltpu.make_async_copy(types_hbm.at[pl.ds(base, TPW)], types_v, sem_in).wait()
    pltpu.make_async_copy(values_hbm.at[pl.ds(base, TPW)], values_v, sem_in).wait()

    for r in range(5):
        for j in range(8):
            sl = pl.ds(16 * j, 16)
            table_v[r, sl] = table_v[r, sl] + b_v[sl]

    w_parts = [w_v[pl.ds(16 * j, 16)] for j in range(8)]

    rows_bufs = (rows0_v, rows1_v)
    sems = (sem0, sem1)

    def compute_chunk(ci, rows_buf):
        @plsc.parallel_loop(0, CHUNK // 16, unroll=2)
        def grp_body(g):
            i0 = ci * CHUNK + g * 16
            tvec = types_v[pl.ds(i0, 16)]
            vvec = values_v[pl.ds(i0, 16)]
            ts = [tvec[k] for k in range(16)]
            vws = [[vvec[k] * w_parts[j] for j in range(8)]
                   for k in range(16)]
            for k in range(16):
                rows = [table_v[ts[k], pl.ds(16 * j, 16)] for j in range(8)]
                for j in range(8):
                    rows_buf[g * 16 + k, pl.ds(16 * j, 16)] = (
                        rows[j] + vws[k][j])

    def pair_body(p, carry):
        for b in range(2):
            ci = p * 2 + b

            @pl.when(p > 0)
            def _wait():
                pltpu.make_async_copy(
                    rows_bufs[b],
                    out_hbm.at[pl.ds(base + ci * CHUNK, CHUNK)],
                    sems[b],
                ).wait()

            compute_chunk(ci, rows_bufs[b])
            pltpu.async_copy(
                rows_bufs[b],
                out_hbm.at[pl.ds(base + ci * CHUNK, CHUNK)],
                sems[b],
            )
        return carry

    lax.fori_loop(0, NCHUNKS // 2, pair_body, 0, unroll=False)

    for b in range(2):
        pltpu.make_async_copy(
            rows_bufs[b],
            out_hbm.at[pl.ds(base, CHUNK)],
            sems[b],
        ).wait()


@jax.jit
def kernel(token_types, token_values, token_embedding, value_W, value_b):
    w = value_W[:, 0]
    values = token_values[:, 0]

    mesh = plsc.VectorSubcoreMesh(core_axis_name="c", subcore_axis_name="s")
    sc_fn = pl.kernel(
        _sc_body,
        mesh=mesh,
        out_type=jax.ShapeDtypeStruct((L, D), jnp.float32),
        scratch_types=[
            pltpu.VMEM((TPW,), jnp.int32),
            pltpu.VMEM((TPW,), jnp.float32),
            pltpu.VMEM((CHUNK, D), jnp.float32),
            pltpu.VMEM((CHUNK, D), jnp.float32),
            pltpu.VMEM((5, D), jnp.float32),
            pltpu.VMEM((D,), jnp.float32),
            pltpu.VMEM((D,), jnp.float32),
            pltpu.SemaphoreType.DMA,
            pltpu.SemaphoreType.DMA,
            pltpu.SemaphoreType.DMA,
        ],
    )
    return sc_fn(token_types, values, token_embedding, w, value_b)


# speedup vs baseline: 2.5978x; 1.0523x over previous
import jax
import jax.numpy as jnp
from jax import lax
from jax.experimental import pallas as pl
from jax.experimental.pallas import tpu as pltpu
from jax.experimental.pallas import tpu_sc as plsc

L = 327680
D = 128
NC = 2
NS = 16
NW = NC * NS
TPW = L // NW
CHUNK = 256
NCHUNKS = TPW // CHUNK


def _sc_body(types_hbm, values_hbm, table_hbm, w_hbm, b_hbm, out_hbm,
             types_v, values_v, rows0_v, rows1_v, table_v, w_v, b_v,
             sem0, sem1, sem_in):
    wid = lax.axis_index("s") * NC + lax.axis_index("c")
    base = wid * TPW

    pltpu.async_copy(table_hbm, table_v, sem_in)
    pltpu.async_copy(w_hbm, w_v, sem_in)
    pltpu.async_copy(b_hbm, b_v, sem_in)
    pltpu.async_copy(types_hbm.at[pl.ds(base, TPW)], types_v, sem_in)
    pltpu.async_copy(values_hbm.at[pl.ds(base, TPW)], values_v, sem_in)
    pltpu.make_async_copy(table_hbm, table_v, sem_in).wait()
    pltpu.make_async_copy(w_hbm, w_v, sem_in).wait()
    pltpu.make_async_copy(b_hbm, b_v, sem_in).wait()
    pltpu.make_async_copy(types_hbm.at[pl.ds(base, TPW)], types_v, sem_in).wait()
    pltpu.make_async_copy(values_hbm.at[pl.ds(base, TPW)], values_v, sem_in).wait()

    for r in range(5):
        for j in range(8):
            sl = pl.ds(16 * j, 16)
            table_v[r, sl] = table_v[r, sl] + b_v[sl]

    w_parts = [w_v[pl.ds(16 * j, 16)] for j in range(8)]

    rows_bufs = (rows0_v, rows1_v)
    sems = (sem0, sem1)

    def compute_chunk(ci, rows_buf):
        @plsc.parallel_loop(0, CHUNK // 16, unroll=1)
        def grp_body(g):
            i0 = ci * CHUNK + g * 16
            tvec = types_v[pl.ds(i0, 16)]
            vvec = values_v[pl.ds(i0, 16)]
            ts = [tvec[k] for k in range(16)]
            vws = [[vvec[k] * w_parts[j] for j in range(8)]
                   for k in range(16)]
            for k in range(16):
                rows = [table_v[ts[k], pl.ds(16 * j, 16)] for j in range(8)]
                for j in range(8):
                    rows_buf[g * 16 + k, pl.ds(16 * j, 16)] = (
                        rows[j] + vws[k][j])

    def pair_body(p, carry):
        for b in range(2):
            ci = p * 2 + b

            @pl.when(p > 0)
            def _wait():
                pltpu.make_async_copy(
                    rows_bufs[b],
                    out_hbm.at[pl.ds(base + ci * CHUNK, CHUNK)],
                    sems[b],
                ).wait()

            compute_chunk(ci, rows_bufs[b])
            pltpu.async_copy(
                rows_bufs[b],
                out_hbm.at[pl.ds(base + ci * CHUNK, CHUNK)],
                sems[b],
            )
        return carry

    lax.fori_loop(0, NCHUNKS // 2, pair_body, 0, unroll=False)

    for b in range(2):
        pltpu.make_async_copy(
            rows_bufs[b],
            out_hbm.at[pl.ds(base, CHUNK)],
            sems[b],
        ).wait()


@jax.jit
def kernel(token_types, token_values, token_embedding, value_W, value_b):
    w = value_W[:, 0]
    values = token_values[:, 0]

    mesh = plsc.VectorSubcoreMesh(core_axis_name="c", subcore_axis_name="s")
    sc_fn = pl.kernel(
        _sc_body,
        mesh=mesh,
        out_type=jax.ShapeDtypeStruct((L, D), jnp.float32),
        scratch_types=[
            pltpu.VMEM((TPW,), jnp.int32),
            pltpu.VMEM((TPW,), jnp.float32),
            pltpu.VMEM((CHUNK, D), jnp.float32),
            pltpu.VMEM((CHUNK, D), jnp.float32),
            pltpu.VMEM((5, D), jnp.float32),
            pltpu.VMEM((D,), jnp.float32),
            pltpu.VMEM((D,), jnp.float32),
            pltpu.SemaphoreType.DMA,
            pltpu.SemaphoreType.DMA,
            pltpu.SemaphoreType.DMA,
        ],
    )
    return sc_fn(token_types, values, token_embedding, w, value_b)
